# baseline (speedup 1.0000x reference)
import functools
import math

import jax
import jax.numpy as jnp
from jax import lax
from jax.experimental import pallas as pl
from jax.experimental.pallas import tpu as pltpu
from jax.experimental.pallas import tpu_sc as plsc

VOCAB = 1000000
EMB = 64
BATCH = 16384
SEQ = 50
SCALE = math.sqrt(EMB)

NC = 2
NS = 16
NW = NC * NS
TOTAL = BATCH * SEQ
BPW = TOTAL // NW
CHUNK = 512
NCHUNK = BPW // CHUNK
ROWS_PER_ITER = 8
VPR = EMB // 16

_mesh = plsc.VectorSubcoreMesh(core_axis_name="c", subcore_axis_name="s")


@functools.partial(
    pl.kernel,
    mesh=_mesh,
    out_type=jax.ShapeDtypeStruct((TOTAL, EMB), jnp.float32),
    scratch_types=[
        pltpu.VMEM((BPW,), jnp.int32),
        pltpu.VMEM((CHUNK, EMB), jnp.float32),
        pltpu.SemaphoreType.DMA,
    ],
    compiler_params=pltpu.CompilerParams(use_tc_tiling_on_sc=False),
)
def _emb_lookup(tokens_hbm, table_hbm, out_hbm, idx_v, rows_v, sem):
    wid = lax.axis_index("s") * NC + lax.axis_index("c")
    base = wid * BPW
    pltpu.sync_copy(tokens_hbm.at[pl.ds(base, BPW)], idx_v)

    def chunk_body(c, carry):
        cb = c * CHUNK
        pltpu.async_copy(
            table_hbm.at[idx_v.at[pl.ds(cb, CHUNK)]], rows_v, sem
        ).wait()

        def scale_body(i, carry2):
            r0 = i * ROWS_PER_ITER
            for r in range(ROWS_PER_ITER):
                for j in range(VPR):
                    sl = pl.ds(j * 16, 16)
                    rows_v[r0 + r, sl] = rows_v[r0 + r, sl] * SCALE
            return carry2

        lax.fori_loop(0, CHUNK // ROWS_PER_ITER, scale_body, 0)
        pltpu.sync_copy(rows_v, out_hbm.at[pl.ds(base + cb, CHUNK)])
        return carry

    lax.fori_loop(0, NCHUNK, chunk_body, 0)


def kernel(tokens, table):
    flat = tokens.reshape(TOTAL)
    out = _emb_lookup(flat, table)
    return out.reshape(BATCH, SEQ, EMB)


# speedup vs baseline: 1.0634x; 1.0634x over previous
import functools
import math

import jax
import jax.numpy as jnp
from jax import lax
from jax.experimental import pallas as pl
from jax.experimental.pallas import tpu as pltpu
from jax.experimental.pallas import tpu_sc as plsc

VOCAB = 1000000
EMB = 64
BATCH = 16384
SEQ = 50
SCALE = math.sqrt(EMB)

NC = 2
NS = 16
NW = NC * NS
TOTAL = BATCH * SEQ
BPW = TOTAL // NW
CHUNK = 512
NCHUNK = BPW // CHUNK
NBUF = 2
VPR = EMB // 16

_mesh = plsc.VectorSubcoreMesh(core_axis_name="c", subcore_axis_name="s")


@functools.partial(
    pl.kernel,
    mesh=_mesh,
    out_type=jax.ShapeDtypeStruct((TOTAL, EMB), jnp.float32),
    scratch_types=[
        pltpu.VMEM((BPW,), jnp.int32),
        pltpu.VMEM((CHUNK, EMB), jnp.float32),
        pltpu.VMEM((CHUNK, EMB), jnp.float32),
        pltpu.SemaphoreType.DMA,
        pltpu.SemaphoreType.DMA,
        pltpu.SemaphoreType.DMA,
        pltpu.SemaphoreType.DMA,
    ],
    compiler_params=pltpu.CompilerParams(use_tc_tiling_on_sc=False),
)
def _emb_lookup(tokens_hbm, table_hbm, out_hbm, idx_v, rows0, rows1,
                gs0, gs1, os0, os1):
    wid = lax.axis_index("s") * NC + lax.axis_index("c")
    base = wid * BPW
    pltpu.sync_copy(tokens_hbm.at[pl.ds(base, BPW)], idx_v)
    bufs = ((rows0, gs0, os0), (rows1, gs1, os1))

    def start_gather(c, rows, sem):
        pltpu.async_copy(table_hbm.at[idx_v.at[pl.ds(c * CHUNK, CHUNK)]],
                         rows, sem)

    def wait_gather(rows, sem):
        pltpu.make_async_copy(table_hbm.at[idx_v.at[pl.ds(0, CHUNK)]],
                              rows, sem).wait()

    def start_write(c, rows, sem):
        pltpu.async_copy(rows, out_hbm.at[pl.ds(base + c * CHUNK, CHUNK)],
                         sem)

    def wait_write(rows, sem):
        pltpu.make_async_copy(rows, out_hbm.at[pl.ds(base, CHUNK)],
                              sem).wait()

    for b in range(NBUF):
        start_gather(b, bufs[b][0], bufs[b][1])

    def loop_body(it, carry):
        c0 = it * NBUF
        for b in range(NBUF):
            c = c0 + b
            rows, gs, osm = bufs[b]
            wait_gather(rows, gs)

            @plsc.parallel_loop(0, CHUNK, 1, unroll=8)
            def _scale(r):
                for j in range(VPR):
                    sl = pl.ds(j * 16, 16)
                    rows[r, sl] = rows[r, sl] * SCALE

            start_write(c, rows, osm)
        for b in range(NBUF):
            c = c0 + b
            rows, gs, osm = bufs[b]
            nc = c + NBUF

            @pl.when(nc < NCHUNK)
            def _():
                wait_write(rows, osm)
                start_gather(nc, rows, gs)

        return carry

    lax.fori_loop(0, NCHUNK // NBUF, loop_body, 0)
    for b in range(NBUF):
        rows, _, osm = bufs[b]
        wait_write(rows, osm)


def kernel(tokens, table):
    flat = tokens.reshape(TOTAL)
    out = _emb_lookup(flat, table)
    return out.reshape(BATCH, SEQ, EMB)
